# Initial kernel scaffold; baseline (speedup 1.0000x reference)
import jax
import jax.numpy as jnp
from jax.experimental import pallas as pl


def kernel(x, edge_index, edge_weight, W1, b1, g1, be1, W2, b2, g2, be2, W3, b3, g3, be3, Wv1, bv1, Wv2, bv2, Wa1, ba1, Wa2, ba2):
    raise NotImplementedError("write your pallas kernel here")


# speedup vs baseline: 7.2110x; 7.2110x over previous
import functools
import jax
import jax.numpy as jnp
from jax import lax
from jax.experimental import pallas as pl
from jax.experimental.pallas import tpu as pltpu
from jax.experimental.pallas import tpu_sc as plsc

_NC = 2
_NS = 16
_NW = _NC * _NS
_C = 128


def _sc_scatter_rows(y, row2d, col2d, w2d, n_rows, K):
    D = y.shape[1]
    n_acc = -(-n_rows // (_NS * _C)) * (_NS * _C)
    n_per_tile = n_acc // _NS
    mesh = plsc.VectorSubcoreMesh(core_axis_name="c", subcore_axis_name="s")

    @functools.partial(
        pl.kernel,
        out_type=jax.ShapeDtypeStruct((_NC, n_acc, D), jnp.float32),
        mesh=mesh,
        compiler_params=pltpu.CompilerParams(use_tc_tiling_on_sc=False),
        scratch_types=[
            pltpu.VMEM((_C,), jnp.int32),
            pltpu.VMEM((1, _C), jnp.int32),
            pltpu.VMEM((_C,), jnp.float32),
            pltpu.VMEM((_C, D), jnp.float32),
            pltpu.VMEM_SHARED((n_acc, D), jnp.float32),
            pltpu.SemaphoreType.DMA,
        ],
    )
    def k(y_h, row_h, col_h, w_h, out_h, ridx_v, cidx_v, w_v, rows_v, acc_sh, sem):
        cid = lax.axis_index("c")
        sid = lax.axis_index("s")
        wid = sid * _NC + cid
        zero16 = jnp.zeros((16,), jnp.float32)

        def zrow(i, carry):
            for g in range(D // 16):
                rows_v[i, pl.ds(g * 16, 16)] = zero16
            return carry

        lax.fori_loop(0, _C, zrow, 0)

        base_r = sid * n_per_tile
        nfull = n_per_tile // _C
        for t in range(nfull):
            pltpu.sync_copy(rows_v, acc_sh.at[pl.ds(base_r + t * _C, _C)])
        plsc.subcore_barrier()

        def chunk(j, carry):
            ch = wid * K + j
            pltpu.sync_copy(row_h.at[ch], ridx_v)
            pltpu.sync_copy(col_h.at[ch], cidx_v.at[0])
            pltpu.sync_copy(w_h.at[ch], w_v)
            pltpu.async_copy(y_h.at[ridx_v], rows_v, sem).wait()

            def scale(i16, c2):
                w16 = w_v[pl.ds(i16 * 16, 16)]
                for l in range(16):
                    wb = lax.gather(
                        w16, jnp.full((16, 1), l, jnp.int32),
                        lax.GatherDimensionNumbers(
                            offset_dims=(), collapsed_slice_dims=(0,),
                            start_index_map=(0,)),
                        slice_sizes=(1,),
                        mode=lax.GatherScatterMode.PROMISE_IN_BOUNDS)
                    e = i16 * 16 + l
                    for g in range(D // 16):
                        sl = pl.ds(g * 16, 16)
                        rows_v[e, sl] = rows_v[e, sl] * wb
                return c2

            lax.fori_loop(0, _C // 16, scale, 0)
            pltpu.sync_copy(rows_v, acc_sh.at[cidx_v.at[0]], add=True)
            return carry

        lax.fori_loop(0, K, chunk, 0)
        plsc.subcore_barrier()

        for t in range(nfull):
            sl = pl.ds(base_r + t * _C, _C)
            pltpu.sync_copy(acc_sh.at[sl], rows_v)
            pltpu.sync_copy(rows_v, out_h.at[cid, sl])

    return k(y, row2d, col2d, w2d)


def _tc_layer_in(degp, x, W, BN=1000):
    N, F = x.shape
    H = W.shape[1]

    def body(deg_ref, x_ref, W_ref, y_ref, dinv_ref):
        d = (deg_ref[0] + deg_ref[1])[:, 0:1] + 1.0
        dinv = jnp.where(d > 0, lax.rsqrt(d), 0.0)
        xw = jnp.dot(x_ref[...], W_ref[...], preferred_element_type=jnp.float32)
        y_ref[...] = xw * dinv
        dinv_ref[...] = dinv

    return pl.pallas_call(
        body,
        grid=(N // BN,),
        in_specs=[
            pl.BlockSpec((2, BN, 16), lambda i: (0, i, 0)),
            pl.BlockSpec((BN, F), lambda i: (i, 0)),
            pl.BlockSpec((F, H), lambda i: (0, 0)),
        ],
        out_specs=[
            pl.BlockSpec((BN, H), lambda i: (i, 0)),
            pl.BlockSpec((BN, 1), lambda i: (i, 0)),
        ],
        out_shape=[
            jax.ShapeDtypeStruct((N, H), jnp.float32),
            jax.ShapeDtypeStruct((N, 1), jnp.float32),
        ],
    )(degp, x, W)


def _ln_relu(conv, mu_g, be):
    mu = jnp.mean(conv, axis=-1, keepdims=True)
    var = jnp.mean((conv - mu) ** 2, axis=-1, keepdims=True)
    h = (conv - mu) * lax.rsqrt(var + 1e-5) * mu_g + be
    return jnp.maximum(h, 0.0)


def _tc_layer_mid(sp, y, dinv, b, g, be, Wn, BN=1000):
    N, H = y.shape
    Hn = Wn.shape[1]

    def body(s_ref, y_ref, dinv_ref, b_ref, g_ref, be_ref, W_ref, out_ref):
        conv = dinv_ref[...] * (s_ref[0] + s_ref[1] + y_ref[...]) + b_ref[...]
        h = _ln_relu(conv, g_ref[...], be_ref[...])
        out_ref[...] = dinv_ref[...] * jnp.dot(
            h, W_ref[...], preferred_element_type=jnp.float32)

    return pl.pallas_call(
        body,
        grid=(N // BN,),
        in_specs=[
            pl.BlockSpec((2, BN, H), lambda i: (0, i, 0)),
            pl.BlockSpec((BN, H), lambda i: (i, 0)),
            pl.BlockSpec((BN, 1), lambda i: (i, 0)),
            pl.BlockSpec((1, H), lambda i: (0, 0)),
            pl.BlockSpec((1, H), lambda i: (0, 0)),
            pl.BlockSpec((1, H), lambda i: (0, 0)),
            pl.BlockSpec((H, Hn), lambda i: (0, 0)),
        ],
        out_specs=pl.BlockSpec((BN, Hn), lambda i: (i, 0)),
        out_shape=jax.ShapeDtypeStruct((N, Hn), jnp.float32),
    )(sp, y, dinv, b, g, be, Wn)


def _tc_layer_final(sp, y, dinv, b, g, be, Wa1, ba1, Wa2, ba2, BN=1000):
    N, H = y.shape

    def body(s_ref, y_ref, dinv_ref, b_ref, g_ref, be_ref, Wa1_ref, ba1_ref,
             Wa2_ref, ba2_ref, adv_ref, sh_ref, sa_ref):
        i = pl.program_id(0)
        conv = dinv_ref[...] * (s_ref[0] + s_ref[1] + y_ref[...]) + b_ref[...]
        h = _ln_relu(conv, g_ref[...], be_ref[...])
        a1 = jnp.maximum(
            jnp.dot(h, Wa1_ref[...], preferred_element_type=jnp.float32)
            + ba1_ref[...], 0.0)
        advb = jnp.dot(a1, Wa2_ref[...], preferred_element_type=jnp.float32) \
            + ba2_ref[...]
        adv_ref[...] = advb

        @pl.when(i == 0)
        def _():
            sh_ref[...] = jnp.zeros_like(sh_ref)
            sa_ref[...] = jnp.zeros_like(sa_ref)

        sh_ref[...] += jnp.sum(h, axis=0, keepdims=True)
        sa_ref[...] += jnp.sum(advb, axis=0, keepdims=True)

    return pl.pallas_call(
        body,
        grid=(N // BN,),
        in_specs=[
            pl.BlockSpec((2, BN, H), lambda i: (0, i, 0)),
            pl.BlockSpec((BN, H), lambda i: (i, 0)),
            pl.BlockSpec((BN, 1), lambda i: (i, 0)),
            pl.BlockSpec((1, H), lambda i: (0, 0)),
            pl.BlockSpec((1, H), lambda i: (0, 0)),
            pl.BlockSpec((1, H), lambda i: (0, 0)),
            pl.BlockSpec((H, 64), lambda i: (0, 0)),
            pl.BlockSpec((1, 64), lambda i: (0, 0)),
            pl.BlockSpec((64, 1), lambda i: (0, 0)),
            pl.BlockSpec((1, 1), lambda i: (0, 0)),
        ],
        out_specs=[
            pl.BlockSpec((BN, 1), lambda i: (i, 0)),
            pl.BlockSpec((1, 64), lambda i: (0, 0)),
            pl.BlockSpec((1, 1), lambda i: (0, 0)),
        ],
        out_shape=[
            jax.ShapeDtypeStruct((N, 1), jnp.float32),
            jax.ShapeDtypeStruct((1, 64), jnp.float32),
            jax.ShapeDtypeStruct((1, 1), jnp.float32),
        ],
    )(sp, y, dinv, b, g, be, Wa1, ba1, Wa2, ba2)


def _tc_head(adv, sh, sa, Wv1, bv1, Wv2, bv2, n, BN=1000):
    N = adv.shape[0]
    inv_n = 1.0 / float(n)

    def body(adv_ref, sh_ref, sa_ref, Wv1_ref, bv1_ref, Wv2_ref, bv2_ref, q_ref):
        hbar = sh_ref[...] * inv_n
        v1 = jnp.maximum(
            jnp.dot(hbar, Wv1_ref[...], preferred_element_type=jnp.float32)
            + bv1_ref[...], 0.0)
        v = jnp.dot(v1, Wv2_ref[...], preferred_element_type=jnp.float32) \
            + bv2_ref[...]
        q_ref[...] = adv_ref[...] + (v - sa_ref[...] * inv_n)

    return pl.pallas_call(
        body,
        grid=(N // BN,),
        in_specs=[
            pl.BlockSpec((BN, 1), lambda i: (i, 0)),
            pl.BlockSpec((1, 64), lambda i: (0, 0)),
            pl.BlockSpec((1, 1), lambda i: (0, 0)),
            pl.BlockSpec((64, 64), lambda i: (0, 0)),
            pl.BlockSpec((1, 64), lambda i: (0, 0)),
            pl.BlockSpec((64, 1), lambda i: (0, 0)),
            pl.BlockSpec((1, 1), lambda i: (0, 0)),
        ],
        out_specs=pl.BlockSpec((BN, 1), lambda i: (i, 0)),
        out_shape=jax.ShapeDtypeStruct((N, 1), jnp.float32),
    )(adv, sh, sa, Wv1, bv1, Wv2, bv2)


def kernel(x, edge_index, edge_weight, W1, b1, g1, be1, W2, b2, g2, be2,
           W3, b3, g3, be3, Wv1, bv1, Wv2, bv2, Wa1, ba1, Wa2, ba2):
    N, F = x.shape
    E = edge_weight.shape[0]
    row = edge_index[0]
    col = edge_index[1]

    K = -(-E // (_NW * _C))
    pad = _NW * K * _C - E
    row2d = jnp.pad(row, (0, pad)).reshape(_NW * K, _C)
    col2d = jnp.pad(col, (0, pad)).reshape(_NW * K, _C)
    w2d = jnp.pad(edge_weight, (0, pad)).reshape(_NW * K, _C)

    ones16 = jnp.ones((N, 16), jnp.float32)
    degp = _sc_scatter_rows(ones16, row2d, col2d, w2d, N, K)

    y1, dinv = _tc_layer_in(degp, x, W1)
    s1 = _sc_scatter_rows(y1, row2d, col2d, w2d, N, K)
    y2 = _tc_layer_mid(s1, y1, dinv, b1.reshape(1, -1), g1.reshape(1, -1),
                       be1.reshape(1, -1), W2)
    s2 = _sc_scatter_rows(y2, row2d, col2d, w2d, N, K)
    y3 = _tc_layer_mid(s2, y2, dinv, b2.reshape(1, -1), g2.reshape(1, -1),
                       be2.reshape(1, -1), W3)
    s3 = _sc_scatter_rows(y3, row2d, col2d, w2d, N, K)
    adv, sh, sa = _tc_layer_final(
        s3, y3, dinv, b3.reshape(1, -1), g3.reshape(1, -1), be3.reshape(1, -1),
        Wa1, ba1.reshape(1, -1), Wa2, ba2.reshape(1, -1))
    q = _tc_head(adv, sh, sa, Wv1, bv1.reshape(1, -1), Wv2, bv2.reshape(1, -1), N)
    return q.reshape(N)
